# Initial kernel scaffold; baseline (speedup 1.0000x reference)
import jax
import jax.numpy as jnp
from jax.experimental import pallas as pl


def kernel(x):
    raise NotImplementedError("write your pallas kernel here")


# speedup vs baseline: 8.0641x; 8.0641x over previous
import jax
import jax.numpy as jnp
from jax.experimental import pallas as pl
from jax.experimental.pallas import tpu as pltpu

_L = 256
_ROWS_PER_CHUNK = 16


def _glcm_frame_kernel(x_ref, out_ref):
    xb = x_ref[0, :, 0, :, :]
    h, w = xb.shape[1], xb.shape[2]
    gray = (xb[0] + xb[1] + xb[2]) / 3.0
    gf = jnp.clip(jnp.floor(gray * 255.0), 0.0, 255.0)
    g = gf.astype(jnp.bfloat16)

    npix = float(h * w)
    mean = jnp.sum(gf) / npix
    var = jnp.sum((gf - mean) * (gf - mean)) / npix
    std = jnp.sqrt(var)

    scol = jnp.full((h, 1), -1, jnp.bfloat16)
    srow = jnp.full((1, w), -1, jnp.bfloat16)
    gl = jnp.concatenate([g[:, 1:], scol], axis=1)
    gd = jnp.concatenate([g[1:, :], srow], axis=0)
    gdl = jnp.concatenate([gd[:, 1:], scol], axis=1)
    gdr = jnp.concatenate([scol, gd[:, :-1]], axis=1)

    r = _ROWS_PER_CHUNK
    nchunks = h // r
    p = r * w

    iota_bf = jax.lax.broadcasted_iota(
        jnp.int32, (r, w, _L), 2).astype(jnp.bfloat16)

    def onehot(arr):
        cmp = arr[:, :, None] == iota_bf
        oh = jnp.where(cmp, jnp.bfloat16(1.0), jnp.bfloat16(0.0))
        return oh.reshape(p, _L)

    dn = (((0,), (0,)), ((), ()))
    h1 = jnp.zeros((_L, _L), jnp.float32)
    h2 = jnp.zeros((_L, _L), jnp.float32)
    h3 = jnp.zeros((_L, _L), jnp.float32)
    h4 = jnp.zeros((_L, _L), jnp.float32)
    for c in range(nchunks):
        rows = slice(c * r, (c + 1) * r)
        oa = onehot(g[rows])
        ol = onehot(gl[rows])
        od = onehot(gd[rows])
        odl = onehot(gdl[rows])
        odr = onehot(gdr[rows])
        h1 = h1 + jax.lax.dot_general(oa, ol, dn,
                                      preferred_element_type=jnp.float32)
        h2 = h2 + jax.lax.dot_general(oa, odl, dn,
                                      preferred_element_type=jnp.float32)
        h3 = h3 + jax.lax.dot_general(oa, od, dn,
                                      preferred_element_type=jnp.float32)
        h4 = h4 + jax.lax.dot_general(oa, odr, dn,
                                      preferred_element_type=jnp.float32)

    ii = jax.lax.broadcasted_iota(jnp.int32, (_L, _L), 0)
    jj = jax.lax.broadcasted_iota(jnp.int32, (_L, _L), 1)
    d = (ii - jj).astype(jnp.float32)
    wcon = d * d
    wdis = jnp.abs(d)
    whom = 1.0 / (1.0 + d * d)

    n_intra = float(h * (w - 1))
    n_diag = float((h - 1) * (w - 1))
    con = jnp.float32(0.0)
    dis = jnp.float32(0.0)
    hom = jnp.float32(0.0)
    asm_ = jnp.float32(0.0)
    for hist, n in ((h1, n_intra), (h2, n_diag), (h3, n_intra), (h4, n_diag)):
        con = con + jnp.sum(hist * wcon) / n
        dis = dis + jnp.sum(hist * wdis) / n
        hom = hom + jnp.sum(hist * whom) / n
        hs = hist + hist.T
        asm_ = asm_ + jnp.sum(hs * hs) / (4.0 * n * n)
    con = con / 4.0
    dis = dis / 4.0
    hom = hom / 4.0
    asm_ = asm_ / 4.0
    energy = jnp.sqrt(asm_)

    out_ref[0, 0, 0] = std
    out_ref[0, 0, 1] = con
    out_ref[0, 0, 2] = dis
    out_ref[0, 0, 3] = hom
    out_ref[0, 0, 4] = asm_
    out_ref[0, 0, 5] = energy


def kernel(x):
    b, c, f, h, w = x.shape
    feats = pl.pallas_call(
        _glcm_frame_kernel,
        grid=(b * f,),
        in_specs=[pl.BlockSpec((1, c, 1, h, w),
                               lambda i, f=f: (i // f, 0, i % f, 0, 0))],
        out_specs=pl.BlockSpec((1, 1, 6), lambda i: (i, 0, 0),
                               memory_space=pltpu.SMEM),
        out_shape=jax.ShapeDtypeStruct((b * f, 1, 6), jnp.float32),
        compiler_params=pltpu.CompilerParams(
            dimension_semantics=("parallel",),
        ),
    )(x)
    return feats.reshape(b, f * 6).astype(jnp.float32)


# speedup vs baseline: 10.7294x; 1.3305x over previous
import jax
import jax.numpy as jnp
from jax.experimental import pallas as pl
from jax.experimental.pallas import tpu as pltpu

_L = 256
_ROWS_PER_CHUNK = 16


def _glcm_frame_kernel(x_ref, out_ref):
    xb = x_ref[0, :, 0, :, :]
    h, w = xb.shape[1], xb.shape[2]
    gray = (xb[0] + xb[1] + xb[2]) / 3.0
    gf = jnp.clip(jnp.floor(gray * 255.0), 0.0, 255.0)
    g = gf.astype(jnp.bfloat16)

    npix = float(h * w)
    mean = jnp.sum(gf) / npix
    var = jnp.sum((gf - mean) * (gf - mean)) / npix
    std = jnp.sqrt(var)

    scol = jnp.full((h, 1), -1, jnp.bfloat16)
    srow = jnp.full((1, w), -1, jnp.bfloat16)
    gl = jnp.concatenate([g[:, 1:], scol], axis=1)
    gr = jnp.concatenate([scol, g[:, :-1]], axis=1)
    gp = jnp.concatenate([g, srow], axis=0)
    glp = jnp.concatenate([gl, srow], axis=0)
    grp = jnp.concatenate([gr, srow], axis=0)

    r = _ROWS_PER_CHUNK
    nchunks = h // r
    p = r * w
    pw = (r + 1) * w

    iota_bf = jax.lax.broadcasted_iota(
        jnp.int32, (r + 1, w, _L), 2).astype(jnp.bfloat16)

    def onehot(arr):
        cmp = arr[:, :, None] == iota_bf
        oh = jnp.where(cmp, jnp.bfloat16(1.0), jnp.bfloat16(0.0))
        return oh.reshape(pw, _L)

    dn = (((0,), (0,)), ((), ()))
    h1 = jnp.zeros((_L, _L), jnp.float32)
    h2 = jnp.zeros((_L, _L), jnp.float32)
    h3 = jnp.zeros((_L, _L), jnp.float32)
    h4 = jnp.zeros((_L, _L), jnp.float32)
    for c in range(nchunks):
        rows = slice(c * r, c * r + r + 1)
        ow = onehot(gp[rows])
        owl = onehot(glp[rows])
        owr = onehot(grp[rows])
        oa = ow[0:p]
        h1 = h1 + jax.lax.dot_general(oa, owl[0:p], dn,
                                      preferred_element_type=jnp.float32)
        h2 = h2 + jax.lax.dot_general(oa, owl[w:pw], dn,
                                      preferred_element_type=jnp.float32)
        h3 = h3 + jax.lax.dot_general(oa, ow[w:pw], dn,
                                      preferred_element_type=jnp.float32)
        h4 = h4 + jax.lax.dot_general(oa, owr[w:pw], dn,
                                      preferred_element_type=jnp.float32)

    ii = jax.lax.broadcasted_iota(jnp.int32, (_L, _L), 0)
    jj = jax.lax.broadcasted_iota(jnp.int32, (_L, _L), 1)
    d = (ii - jj).astype(jnp.float32)
    wcon = d * d
    wdis = jnp.abs(d)
    whom = 1.0 / (1.0 + d * d)

    n_intra = float(h * (w - 1))
    n_diag = float((h - 1) * (w - 1))
    ht = (h1 + h3) / n_intra + (h2 + h4) / n_diag
    con = jnp.sum(ht * wcon) / 4.0
    dis = jnp.sum(ht * wdis) / 4.0
    hom = jnp.sum(ht * whom) / 4.0
    asm_ = jnp.float32(0.0)
    for hist, n in ((h1, n_intra), (h2, n_diag), (h3, n_intra), (h4, n_diag)):
        hs = hist + hist.T
        asm_ = asm_ + jnp.sum(hs * hs) / (4.0 * n * n)
    asm_ = asm_ / 4.0
    energy = jnp.sqrt(asm_)

    out_ref[0, 0, 0] = std
    out_ref[0, 0, 1] = con
    out_ref[0, 0, 2] = dis
    out_ref[0, 0, 3] = hom
    out_ref[0, 0, 4] = asm_
    out_ref[0, 0, 5] = energy


def kernel(x):
    b, c, f, h, w = x.shape
    feats = pl.pallas_call(
        _glcm_frame_kernel,
        grid=(b * f,),
        in_specs=[pl.BlockSpec((1, c, 1, h, w),
                               lambda i, f=f: (i // f, 0, i % f, 0, 0))],
        out_specs=pl.BlockSpec((1, 1, 6), lambda i: (i, 0, 0),
                               memory_space=pltpu.SMEM),
        out_shape=jax.ShapeDtypeStruct((b * f, 1, 6), jnp.float32),
        compiler_params=pltpu.CompilerParams(
            dimension_semantics=("parallel",),
        ),
    )(x)
    return feats.reshape(b, f * 6).astype(jnp.float32)


# speedup vs baseline: 16.7342x; 1.5597x over previous
import jax
import jax.numpy as jnp
from jax.experimental import pallas as pl
from jax.experimental.pallas import tpu as pltpu

_L = 256
_ROWS_PER_CHUNK = 16


def _glcm_frame_kernel(x_ref, out_ref):
    xb = x_ref[0, :, 0, :, :]
    h, w = xb.shape[1], xb.shape[2]
    gray = (xb[0] + xb[1] + xb[2]) / 3.0
    gf = jnp.clip(jnp.floor(gray * 255.0), 0.0, 255.0)
    g = gf.astype(jnp.bfloat16)

    npix = float(h * w)
    mean = jnp.sum(gf) / npix
    var = jnp.sum((gf - mean) * (gf - mean)) / npix
    std = jnp.sqrt(var)

    scol = jnp.full((h, 1), -1, jnp.bfloat16)
    srow = jnp.full((1, w), -1, jnp.bfloat16)
    gl = jnp.concatenate([g[:, 1:], scol], axis=1)
    gr = jnp.concatenate([scol, g[:, :-1]], axis=1)
    gp = jnp.concatenate([g, srow], axis=0)
    glp = jnp.concatenate([gl, srow], axis=0)
    grp = jnp.concatenate([gr, srow], axis=0)

    r = _ROWS_PER_CHUNK
    nchunks = h // r
    p = r * w
    pw = (r + 1) * w

    iota_bf = jax.lax.broadcasted_iota(
        jnp.int32, (r + 1, w, _L), 2).astype(jnp.bfloat16)

    def onehot(arr):
        cmp = arr[:, :, None] == iota_bf
        oh = jnp.where(cmp, jnp.bfloat16(1.0), jnp.bfloat16(0.0))
        return oh.astype(jnp.float8_e4m3fn).reshape(pw, _L)

    dn = (((0,), (0,)), ((), ()))
    h1 = jnp.zeros((_L, _L), jnp.float32)
    h2 = jnp.zeros((_L, _L), jnp.float32)
    h3 = jnp.zeros((_L, _L), jnp.float32)
    h4 = jnp.zeros((_L, _L), jnp.float32)
    for c in range(nchunks):
        rows = slice(c * r, c * r + r + 1)
        ow = onehot(gp[rows])
        owl = onehot(glp[rows])
        owr = onehot(grp[rows])
        oa = ow[0:p]
        h1 = h1 + jax.lax.dot_general(oa, owl[0:p], dn,
                                      preferred_element_type=jnp.float32)
        h2 = h2 + jax.lax.dot_general(oa, owl[w:pw], dn,
                                      preferred_element_type=jnp.float32)
        h3 = h3 + jax.lax.dot_general(oa, ow[w:pw], dn,
                                      preferred_element_type=jnp.float32)
        h4 = h4 + jax.lax.dot_general(oa, owr[w:pw], dn,
                                      preferred_element_type=jnp.float32)

    ii = jax.lax.broadcasted_iota(jnp.int32, (_L, _L), 0)
    jj = jax.lax.broadcasted_iota(jnp.int32, (_L, _L), 1)
    d = (ii - jj).astype(jnp.float32)
    wcon = d * d
    wdis = jnp.abs(d)
    whom = 1.0 / (1.0 + d * d)

    n_intra = float(h * (w - 1))
    n_diag = float((h - 1) * (w - 1))
    ht = (h1 + h3) / n_intra + (h2 + h4) / n_diag
    con = jnp.sum(ht * wcon) / 4.0
    dis = jnp.sum(ht * wdis) / 4.0
    hom = jnp.sum(ht * whom) / 4.0
    asm_ = jnp.float32(0.0)
    for hist, n in ((h1, n_intra), (h2, n_diag), (h3, n_intra), (h4, n_diag)):
        hs = hist + hist.T
        asm_ = asm_ + jnp.sum(hs * hs) / (4.0 * n * n)
    asm_ = asm_ / 4.0
    energy = jnp.sqrt(asm_)

    out_ref[0, 0, 0] = std
    out_ref[0, 0, 1] = con
    out_ref[0, 0, 2] = dis
    out_ref[0, 0, 3] = hom
    out_ref[0, 0, 4] = asm_
    out_ref[0, 0, 5] = energy


def kernel(x):
    b, c, f, h, w = x.shape
    feats = pl.pallas_call(
        _glcm_frame_kernel,
        grid=(b * f,),
        in_specs=[pl.BlockSpec((1, c, 1, h, w),
                               lambda i, f=f: (i // f, 0, i % f, 0, 0))],
        out_specs=pl.BlockSpec((1, 1, 6), lambda i: (i, 0, 0),
                               memory_space=pltpu.SMEM),
        out_shape=jax.ShapeDtypeStruct((b * f, 1, 6), jnp.float32),
        compiler_params=pltpu.CompilerParams(
            dimension_semantics=("parallel",),
        ),
    )(x)
    return feats.reshape(b, f * 6).astype(jnp.float32)


# speedup vs baseline: 16.7810x; 1.0028x over previous
import jax
import jax.numpy as jnp
from jax.experimental import pallas as pl
from jax.experimental.pallas import tpu as pltpu

_L = 256
_ROWS_PER_CHUNK = 16


def _frame_feats_body(xb, out_ref, sub):
    h, w = xb.shape[1], xb.shape[2]
    gray = (xb[0] + xb[1] + xb[2]) / 3.0
    gf = jnp.clip(jnp.floor(gray * 255.0), 0.0, 255.0)
    g = gf.astype(jnp.bfloat16)

    npix = float(h * w)
    mean = jnp.sum(gf) / npix
    var = jnp.sum((gf - mean) * (gf - mean)) / npix
    std = jnp.sqrt(var)

    scol = jnp.full((h, 1), -1, jnp.bfloat16)
    srow = jnp.full((1, w), -1, jnp.bfloat16)
    gl = jnp.concatenate([g[:, 1:], scol], axis=1)
    gr = jnp.concatenate([scol, g[:, :-1]], axis=1)
    gp = jnp.concatenate([g, srow], axis=0)
    glp = jnp.concatenate([gl, srow], axis=0)
    grp = jnp.concatenate([gr, srow], axis=0)

    r = _ROWS_PER_CHUNK
    nchunks = h // r
    p = r * w
    pw = (r + 1) * w

    iota_bf = jax.lax.broadcasted_iota(
        jnp.int32, (r + 1, w, _L), 2).astype(jnp.bfloat16)

    def onehot(arr):
        cmp = arr[:, :, None] == iota_bf
        oh = jnp.where(cmp, jnp.bfloat16(1.0), jnp.bfloat16(0.0))
        return oh.astype(jnp.float8_e4m3fn).reshape(pw, _L)

    dn = (((0,), (0,)), ((), ()))
    h1 = jnp.zeros((_L, _L), jnp.float32)
    h2 = jnp.zeros((_L, _L), jnp.float32)
    h3 = jnp.zeros((_L, _L), jnp.float32)
    h4 = jnp.zeros((_L, _L), jnp.float32)
    for c in range(nchunks):
        rows = slice(c * r, c * r + r + 1)
        ow = onehot(gp[rows])
        owl = onehot(glp[rows])
        owr = onehot(grp[rows])
        oa = ow[0:p]
        h1 = h1 + jax.lax.dot_general(oa, owl[0:p], dn,
                                      preferred_element_type=jnp.float32)
        h2 = h2 + jax.lax.dot_general(oa, owl[w:pw], dn,
                                      preferred_element_type=jnp.float32)
        h3 = h3 + jax.lax.dot_general(oa, ow[w:pw], dn,
                                      preferred_element_type=jnp.float32)
        h4 = h4 + jax.lax.dot_general(oa, owr[w:pw], dn,
                                      preferred_element_type=jnp.float32)

    ii = jax.lax.broadcasted_iota(jnp.int32, (_L, _L), 0)
    jj = jax.lax.broadcasted_iota(jnp.int32, (_L, _L), 1)
    d = (ii - jj).astype(jnp.float32)
    wcon = d * d
    wdis = jnp.abs(d)
    whom = 1.0 / (1.0 + d * d)

    n_intra = float(h * (w - 1))
    n_diag = float((h - 1) * (w - 1))
    ht = (h1 + h3) / n_intra + (h2 + h4) / n_diag
    con = jnp.sum(ht * wcon) / 4.0
    dis = jnp.sum(ht * wdis) / 4.0
    hom = jnp.sum(ht * whom) / 4.0
    asm_ = jnp.float32(0.0)
    for hist, n in ((h1, n_intra), (h2, n_diag), (h3, n_intra), (h4, n_diag)):
        hs = hist + hist.T
        asm_ = asm_ + jnp.sum(hs * hs) / (4.0 * n * n)
    asm_ = asm_ / 4.0
    energy = jnp.sqrt(asm_)

    out_ref[sub, 0, 0] = std
    out_ref[sub, 0, 1] = con
    out_ref[sub, 0, 2] = dis
    out_ref[sub, 0, 3] = hom
    out_ref[sub, 0, 4] = asm_
    out_ref[sub, 0, 5] = energy


def _glcm_frame_kernel(x_ref, out_ref):
    _frame_feats_body(x_ref[0, :, 0, :, :], out_ref, 0)
    _frame_feats_body(x_ref[0, :, 1, :, :], out_ref, 1)


def kernel(x):
    b, c, f, h, w = x.shape
    feats = pl.pallas_call(
        _glcm_frame_kernel,
        grid=(b * f // 2,),
        in_specs=[pl.BlockSpec((1, c, 2, h, w),
                               lambda i, f=f: (i // (f // 2), 0,
                                               i % (f // 2), 0, 0))],
        out_specs=pl.BlockSpec((2, 1, 6), lambda i: (i, 0, 0),
                               memory_space=pltpu.SMEM),
        out_shape=jax.ShapeDtypeStruct((b * f, 1, 6), jnp.float32),
        compiler_params=pltpu.CompilerParams(
            dimension_semantics=("parallel",),
        ),
    )(x)
    return feats.reshape(b, f * 6).astype(jnp.float32)


# speedup vs baseline: 21.1262x; 1.2589x over previous
import jax
import jax.numpy as jnp
from jax.experimental import pallas as pl
from jax.experimental.pallas import tpu as pltpu

_L = 256
_ROWS_PER_CHUNK = 16


def _frame_feats_body(xb, xb14, out_ref, sub):
    h, w = xb.shape[1], xb.shape[2]
    gray = (xb[0] + xb[1] + xb[2]) / 3.0
    gf = jnp.clip(jnp.floor(gray * 255.0), 0.0, 255.0)
    gray14 = (xb14[0] + xb14[1] + xb14[2]) / 3.0
    g14 = jnp.clip(jnp.floor(gray14 * 255.0), 0.0, 255.0).astype(jnp.bfloat16)
    g = gf.astype(jnp.bfloat16)

    npix = float(h * w)
    mean = jnp.sum(gf) / npix
    var = jnp.sum((gf - mean) * (gf - mean)) / npix
    std = jnp.sqrt(var)

    scol = jnp.full((h, 1), -1, jnp.bfloat16)
    srow = jnp.full((1, w), -1, jnp.bfloat16)
    gl = jnp.concatenate([g[:, 1:], scol], axis=1)
    gr = jnp.concatenate([scol, g[:, :-1]], axis=1)
    gp = jnp.concatenate([g, srow], axis=0)
    glp = jnp.concatenate([gl, srow], axis=0)
    grp = jnp.concatenate([gr, srow], axis=0)

    r = _ROWS_PER_CHUNK
    nchunks = h // r
    p = r * w
    pw = (r + 1) * w

    iota_bf = jax.lax.broadcasted_iota(
        jnp.int32, (r + 1, w, _L), 2).astype(jnp.bfloat16)

    def onehot(arr):
        cmp = arr[:, :, None] == iota_bf
        oh = jnp.where(cmp, jnp.bfloat16(1.0), jnp.bfloat16(0.0))
        return oh.astype(jnp.float8_e4m3fn).reshape(pw, _L)

    iota_sub = jax.lax.broadcasted_iota(
        jnp.int32, (_L, p), 0).astype(jnp.bfloat16)

    def onehot_t(row):
        cmp = row == iota_sub
        oh = jnp.where(cmp, jnp.bfloat16(1.0), jnp.bfloat16(0.0))
        return oh.astype(jnp.float8_e4m3fn)

    dn = (((1,), (0,)), ((), ()))
    h1 = jnp.zeros((_L, _L), jnp.float32)
    h2 = jnp.zeros((_L, _L), jnp.float32)
    h3 = jnp.zeros((_L, _L), jnp.float32)
    h4 = jnp.zeros((_L, _L), jnp.float32)
    for c in range(nchunks):
        rows = slice(c * r, c * r + r + 1)
        ow = onehot(gp[rows])
        owl = onehot(glp[rows])
        owr = onehot(grp[rows])
        oat = onehot_t(g14[c:c + 1, :])
        h1 = h1 + jax.lax.dot_general(oat, owl[0:p], dn,
                                      preferred_element_type=jnp.float32)
        h2 = h2 + jax.lax.dot_general(oat, owl[w:pw], dn,
                                      preferred_element_type=jnp.float32)
        h3 = h3 + jax.lax.dot_general(oat, ow[w:pw], dn,
                                      preferred_element_type=jnp.float32)
        h4 = h4 + jax.lax.dot_general(oat, owr[w:pw], dn,
                                      preferred_element_type=jnp.float32)

    ii = jax.lax.broadcasted_iota(jnp.int32, (_L, _L), 0)
    jj = jax.lax.broadcasted_iota(jnp.int32, (_L, _L), 1)
    d = (ii - jj).astype(jnp.float32)
    wcon = d * d
    wdis = jnp.abs(d)
    whom = 1.0 / (1.0 + d * d)

    n_intra = float(h * (w - 1))
    n_diag = float((h - 1) * (w - 1))
    ht = (h1 + h3) / n_intra + (h2 + h4) / n_diag
    con = jnp.sum(ht * wcon) / 4.0
    dis = jnp.sum(ht * wdis) / 4.0
    hom = jnp.sum(ht * whom) / 4.0
    asm_ = jnp.float32(0.0)
    for hist, n in ((h1, n_intra), (h2, n_diag), (h3, n_intra), (h4, n_diag)):
        hs = hist + hist.T
        asm_ = asm_ + jnp.sum(hs * hs) / (4.0 * n * n)
    asm_ = asm_ / 4.0
    energy = jnp.sqrt(asm_)

    out_ref[sub, 0, 0] = std
    out_ref[sub, 0, 1] = con
    out_ref[sub, 0, 2] = dis
    out_ref[sub, 0, 3] = hom
    out_ref[sub, 0, 4] = asm_
    out_ref[sub, 0, 5] = energy


def _glcm_frame_kernel(x_ref, x14_ref, out_ref):
    _frame_feats_body(x_ref[0, :, 0, :, :], x14_ref[0, :, 0, :, :], out_ref, 0)
    _frame_feats_body(x_ref[0, :, 1, :, :], x14_ref[0, :, 1, :, :], out_ref, 1)


def kernel(x):
    b, c, f, h, w = x.shape
    r = _ROWS_PER_CHUNK
    x14 = x.reshape(b, c, f, h // r, r * w)
    feats = pl.pallas_call(
        _glcm_frame_kernel,
        grid=(b * f // 2,),
        in_specs=[pl.BlockSpec((1, c, 2, h, w),
                               lambda i, f=f: (i // (f // 2), 0,
                                               i % (f // 2), 0, 0)),
                  pl.BlockSpec((1, c, 2, h // r, r * w),
                               lambda i, f=f: (i // (f // 2), 0,
                                               i % (f // 2), 0, 0))],
        out_specs=pl.BlockSpec((2, 1, 6), lambda i: (i, 0, 0),
                               memory_space=pltpu.SMEM),
        out_shape=jax.ShapeDtypeStruct((b * f, 1, 6), jnp.float32),
        compiler_params=pltpu.CompilerParams(
            dimension_semantics=("parallel",),
        ),
    )(x, x14)
    return feats.reshape(b, f * 6).astype(jnp.float32)
